# baseline (speedup 1.0000x reference)
import functools

import jax
import jax.numpy as jnp
from jax import lax
from jax.experimental import pallas as pl
from jax.experimental.pallas import tpu as pltpu
from jax.experimental.pallas import tpu_sc as plsc

N = 10000
E = 320000
D = 128
G = 64
EPS = 1e-5
NEG = 0.01

NC = 2
NS = 16
NW = NC * NS
EB = 128
GPT = 80
E_PAD = NW * GPT * EB
N_PAD = 10112
RPT = N_PAD // NS

_HIGH = jax.lax.Precision.HIGHEST


def _mesh():
    return plsc.VectorSubcoreMesh(core_axis_name="c", subcore_axis_name="s",
                                  num_cores=NC, num_subcores=NS)


def _zero_fill(buf, nrows, ncols):
    z16 = jnp.zeros((16,), jnp.float32)

    @pl.loop(0, nrows)
    def _(r):
        @pl.loop(0, ncols, step=16)
        def _(col):
            buf[r, pl.ds(col, 16)] = z16


def _dma_rows(src_buf, dst_ref, base):
    nfull = RPT // EB
    rem = RPT - nfull * EB
    for k in range(nfull):
        pltpu.sync_copy(src_buf, dst_ref.at[pl.ds(base + k * EB, EB)])
    if rem:
        pltpu.sync_copy(src_buf.at[0:rem], dst_ref.at[pl.ds(base + nfull * EB, rem)])


CH = 40
CPT0 = 1
CPT1 = 3


def _make_sc_agg():
    @functools.partial(
        pl.kernel,
        out_type=jax.ShapeDtypeStruct((NC * N_PAD, D), jnp.float32),
        mesh=_mesh(),
        scratch_types=[
            pltpu.VMEM_SHARED((N_PAD, D), jnp.float32),
            pltpu.VMEM((CH, EB), jnp.int32),
            pltpu.VMEM((CH, EB), jnp.int32),
            [pltpu.VMEM((EB, D), jnp.float32)] * 2,
        ],
    )
    def agg(h_hbm, src_hbm, dst_hbm, out_hbm, acc, idx_s, idx_d, rows):
        c = lax.axis_index("c")
        s = lax.axis_index("s")
        wid = s * NC + c
        base = s * RPT

        _zero_fill(rows[0], EB, D)
        _dma_rows(rows[0], acc, base)
        plsc.subcore_barrier()

        start = jnp.where(c == 0, s * (CPT0 * CH), NS * (CPT0 * CH)
                          + s * (CPT1 * CH))
        nch = jnp.where(c == 0, CPT0, CPT1)

        @functools.partial(pl.run_scoped,
                           gs0=pltpu.SemaphoreType.DMA(()),
                           gs1=pltpu.SemaphoreType.DMA(()))
        def _(gs0, gs1):
            gsem = (gs0, gs1)

            @pl.loop(0, nch)
            def _(q):
                off = pl.multiple_of(start + q * CH, 8)
                pltpu.sync_copy(src_hbm.at[pl.ds(off, CH)], idx_s)
                pltpu.sync_copy(dst_hbm.at[pl.ds(off, CH)], idx_d)
                pltpu.async_copy(h_hbm.at[idx_s.at[0]], rows[0], gsem[0])

                @pl.loop(0, (CH - 2) // 2)
                def _(tt):
                    for b in range(2):
                        j = tt * 2 + b
                        pltpu.async_copy(h_hbm.at[idx_s.at[j + 1]],
                                         rows[1 - b], gsem[1 - b])
                        pltpu.make_async_copy(h_hbm.at[idx_s.at[j]],
                                              rows[b], gsem[b]).wait()
                        pltpu.sync_copy(rows[b], acc.at[idx_d.at[j]],
                                        add=True)

                pltpu.async_copy(h_hbm.at[idx_s.at[CH - 1]], rows[1], gsem[1])
                pltpu.make_async_copy(h_hbm.at[idx_s.at[CH - 2]],
                                      rows[0], gsem[0]).wait()
                pltpu.sync_copy(rows[0], acc.at[idx_d.at[CH - 2]], add=True)
                pltpu.make_async_copy(h_hbm.at[idx_s.at[CH - 1]],
                                      rows[1], gsem[1]).wait()
                pltpu.sync_copy(rows[1], acc.at[idx_d.at[CH - 1]], add=True)

        plsc.subcore_barrier()
        pltpu.sync_copy(acc.at[pl.ds(base, RPT)],
                        out_hbm.at[pl.ds(c * N_PAD + base, RPT)])

    return agg


def _make_sc_degree():
    @functools.partial(
        pl.kernel,
        out_type=jax.ShapeDtypeStruct((NC * N_PAD, 16), jnp.float32),
        mesh=_mesh(),
        scratch_types=[
            pltpu.VMEM_SHARED((N_PAD, 16), jnp.float32),
            pltpu.VMEM((GPT, EB), jnp.int32),
            pltpu.VMEM((EB, 16), jnp.float32),
        ],
    )
    def degree(dst_hbm, out_hbm, acc, idx_d, ones):
        c = lax.axis_index("c")
        s = lax.axis_index("s")
        wid = s * NC + c
        base = s * RPT

        _zero_fill(ones, EB, 16)
        _dma_rows(ones, acc, base)

        e0 = jnp.where(lax.iota(jnp.int32, 16) == 0,
                       jnp.float32(1.0), jnp.float32(0.0))

        @pl.loop(0, EB)
        def _(r):
            ones[r, :] = e0

        pltpu.sync_copy(dst_hbm.at[pl.ds(wid * GPT, GPT)], idx_d)
        plsc.subcore_barrier()

        @pl.loop(0, GPT)
        def _(j):
            pltpu.sync_copy(ones, acc.at[idx_d.at[j]], add=True)

        plsc.subcore_barrier()
        pltpu.sync_copy(acc.at[pl.ds(base, RPT)],
                        out_hbm.at[pl.ds(c * N_PAD + base, RPT)])

    return degree


def _mm_body(x_ref, w_ref, o_ref):
    h = jnp.dot(x_ref[...], w_ref[...], precision=_HIGH,
                preferred_element_type=jnp.float32)
    o_ref[0:N, :] = h
    o_ref[N:N_PAD, :] = jnp.zeros((N_PAD - N, D), jnp.float32)


def _tc_matmul_pad(x, w):
    return pl.pallas_call(
        _mm_body,
        out_shape=jax.ShapeDtypeStruct((N_PAD, D), jnp.float32),
    )(x, w)


def _scale_body(h_ref, degp_ref, hp_ref, dinv_ref):
    deg = (degp_ref[0:N_PAD, 0:1] + degp_ref[N_PAD:2 * N_PAD, 0:1]) + 1.0
    rowid = lax.broadcasted_iota(jnp.int32, (N_PAD, 1), 0)
    dinv = jnp.where(rowid < N, lax.rsqrt(deg), 0.0)
    dinv_ref[...] = dinv
    hp_ref[...] = h_ref[...] * dinv


def _tc_scale(h, degp):
    return pl.pallas_call(
        _scale_body,
        out_shape=(jax.ShapeDtypeStruct((N_PAD, D), jnp.float32),
                   jax.ShapeDtypeStruct((N_PAD, 1), jnp.float32)),
    )(h, degp)


def _post_conv(p_ref, h_ref, dinv_ref, b_ref, g_ref, be_ref):
    hp = h_ref[0:N, :]
    y = (p_ref[0:N, :] + p_ref[N_PAD:N_PAD + N, :] + hp) * dinv_ref[0:N, :]
    y = y + b_ref[...]
    y = jnp.where(y >= 0, y, NEG * y)
    mean = jnp.mean(y, axis=0, keepdims=True)
    cent = y - mean
    var = jnp.mean(cent * cent, axis=0, keepdims=True)
    return cent / jnp.sqrt(var + EPS) * g_ref[...] + be_ref[...]


def _mid_body(p_ref, h_ref, dinv_ref, b_ref, g_ref, be_ref, w_ref, o_ref):
    z = _post_conv(p_ref, h_ref, dinv_ref, b_ref, g_ref, be_ref)
    hn = jnp.dot(z, w_ref[...], precision=_HIGH,
                 preferred_element_type=jnp.float32)
    o_ref[0:N, :] = hn * dinv_ref[0:N, :]
    o_ref[N:N_PAD, :] = jnp.zeros((N_PAD - N, D), jnp.float32)


def _tc_mid(p, h, dinv, b, g, be, w_next):
    return pl.pallas_call(
        _mid_body,
        out_shape=jax.ShapeDtypeStruct((N_PAD, D), jnp.float32),
    )(p, h, dinv, b, g, be, w_next)


def _final_body(p_ref, h_ref, dinv_ref, b_ref, g_ref, be_ref, batch_ref, o_ref):
    z = _post_conv(p_ref, h_ref, dinv_ref, b_ref, g_ref, be_ref)
    onehot = (batch_ref[...] == lax.broadcasted_iota(jnp.int32, (1, G), 1))
    onehot = onehot.astype(jnp.float32)
    sums = jax.lax.dot_general(onehot, z, (((0,), (0,)), ((), ())),
                               precision=_HIGH,
                               preferred_element_type=jnp.float32)
    counts = jnp.sum(onehot, axis=0)[:, None]
    o_ref[...] = sums / jnp.maximum(counts, 1.0)


def _tc_final(p, h, dinv, b, g, be, batch2d):
    return pl.pallas_call(
        _final_body,
        out_shape=jax.ShapeDtypeStruct((G, D), jnp.float32),
    )(p, h, dinv, b, g, be, batch2d)


def kernel(x, edge_index, batch, W1, b1, g1, be1, W2, b2, g2, be2,
           W3, b3, g3, be3):
    src = edge_index[0].astype(jnp.int32)
    dst = edge_index[1].astype(jnp.int32)
    pad = jnp.full((E_PAD - E,), N, jnp.int32)
    src2d = jnp.concatenate([src, pad]).reshape(NW * GPT, EB)
    dst2d = jnp.concatenate([dst, pad]).reshape(NW * GPT, EB)
    batch2d = batch.astype(jnp.int32).reshape(N, 1)
    b1, g1, be1 = b1.reshape(1, D), g1.reshape(1, D), be1.reshape(1, D)
    b2, g2, be2 = b2.reshape(1, D), g2.reshape(1, D), be2.reshape(1, D)
    b3, g3, be3 = b3.reshape(1, D), g3.reshape(1, D), be3.reshape(1, D)

    sc_agg = _make_sc_agg()
    degp = _make_sc_degree()(dst2d)
    h1 = _tc_matmul_pad(x, W1)
    h1p, dinv = _tc_scale(h1, degp)

    p1 = sc_agg(h1p, src2d, dst2d)
    h2p = _tc_mid(p1, h1p, dinv, b1, g1, be1, W2)
    p2 = sc_agg(h2p, src2d, dst2d)
    h3p = _tc_mid(p2, h2p, dinv, b2, g2, be2, W3)
    return _tc_final(sc_agg(h3p, src2d, dst2d), h3p, dinv, b3, g3, be3, batch2d)


# speedup vs baseline: 1.0966x; 1.0966x over previous
import functools

import jax
import jax.numpy as jnp
from jax import lax
from jax.experimental import pallas as pl
from jax.experimental.pallas import tpu as pltpu
from jax.experimental.pallas import tpu_sc as plsc

N = 10000
E = 320000
D = 128
G = 64
EPS = 1e-5
NEG = 0.01

NC = 2
NS = 16
NW = NC * NS
EB = 128
GPT = 80
E_PAD = NW * GPT * EB
N_PAD = 10112
RPT = N_PAD // NS

_HIGH = jax.lax.Precision.HIGHEST


def _mesh():
    return plsc.VectorSubcoreMesh(core_axis_name="c", subcore_axis_name="s",
                                  num_cores=NC, num_subcores=NS)


def _zero_fill(buf, nrows, ncols):
    z16 = jnp.zeros((16,), jnp.float32)

    @pl.loop(0, nrows)
    def _(r):
        @pl.loop(0, ncols, step=16)
        def _(col):
            buf[r, pl.ds(col, 16)] = z16


def _dma_rows(src_buf, dst_ref, base):
    nfull = RPT // EB
    rem = RPT - nfull * EB
    for k in range(nfull):
        pltpu.sync_copy(src_buf, dst_ref.at[pl.ds(base + k * EB, EB)])
    if rem:
        pltpu.sync_copy(src_buf.at[0:rem], dst_ref.at[pl.ds(base + nfull * EB, rem)])


CH = 40
CPT0 = 3
CPT1 = 1


def _make_sc_agg():
    @functools.partial(
        pl.kernel,
        out_type=jax.ShapeDtypeStruct((NC * N_PAD, D), jnp.float32),
        mesh=_mesh(),
        scratch_types=[
            pltpu.VMEM_SHARED((N_PAD, D), jnp.float32),
            pltpu.VMEM((CH, EB), jnp.int32),
            pltpu.VMEM((CH, EB), jnp.int32),
            [pltpu.VMEM((EB, D), jnp.float32)] * 2,
        ],
    )
    def agg(h_hbm, src_hbm, dst_hbm, out_hbm, acc, idx_s, idx_d, rows):
        c = lax.axis_index("c")
        s = lax.axis_index("s")
        wid = s * NC + c
        base = s * RPT

        _zero_fill(rows[0], EB, D)
        _dma_rows(rows[0], acc, base)
        plsc.subcore_barrier()

        start = jnp.where(c == 0, s * (CPT0 * CH), NS * (CPT0 * CH)
                          + s * (CPT1 * CH))
        nch = jnp.where(c == 0, CPT0, CPT1)

        @functools.partial(pl.run_scoped,
                           gs0=pltpu.SemaphoreType.DMA(()),
                           gs1=pltpu.SemaphoreType.DMA(()))
        def _(gs0, gs1):
            gsem = (gs0, gs1)

            @pl.loop(0, nch)
            def _(q):
                off = pl.multiple_of(start + q * CH, 8)
                pltpu.sync_copy(src_hbm.at[pl.ds(off, CH)], idx_s)
                pltpu.sync_copy(dst_hbm.at[pl.ds(off, CH)], idx_d)
                pltpu.async_copy(h_hbm.at[idx_s.at[0]], rows[0], gsem[0])

                @pl.loop(0, (CH - 2) // 2)
                def _(tt):
                    for b in range(2):
                        j = tt * 2 + b
                        pltpu.async_copy(h_hbm.at[idx_s.at[j + 1]],
                                         rows[1 - b], gsem[1 - b])
                        pltpu.make_async_copy(h_hbm.at[idx_s.at[j]],
                                              rows[b], gsem[b]).wait()
                        pltpu.sync_copy(rows[b], acc.at[idx_d.at[j]],
                                        add=True)

                pltpu.async_copy(h_hbm.at[idx_s.at[CH - 1]], rows[1], gsem[1])
                pltpu.make_async_copy(h_hbm.at[idx_s.at[CH - 2]],
                                      rows[0], gsem[0]).wait()
                pltpu.sync_copy(rows[0], acc.at[idx_d.at[CH - 2]], add=True)
                pltpu.make_async_copy(h_hbm.at[idx_s.at[CH - 1]],
                                      rows[1], gsem[1]).wait()
                pltpu.sync_copy(rows[1], acc.at[idx_d.at[CH - 1]], add=True)

        plsc.subcore_barrier()
        pltpu.sync_copy(acc.at[pl.ds(base, RPT)],
                        out_hbm.at[pl.ds(c * N_PAD + base, RPT)])

    return agg


def _make_sc_degree():
    @functools.partial(
        pl.kernel,
        out_type=jax.ShapeDtypeStruct((NC * N_PAD, 16), jnp.float32),
        mesh=_mesh(),
        scratch_types=[
            pltpu.VMEM_SHARED((N_PAD, 16), jnp.float32),
            pltpu.VMEM((GPT, EB), jnp.int32),
            pltpu.VMEM((EB, 16), jnp.float32),
        ],
    )
    def degree(dst_hbm, out_hbm, acc, idx_d, ones):
        c = lax.axis_index("c")
        s = lax.axis_index("s")
        wid = s * NC + c
        base = s * RPT

        _zero_fill(ones, EB, 16)
        _dma_rows(ones, acc, base)

        e0 = jnp.where(lax.iota(jnp.int32, 16) == 0,
                       jnp.float32(1.0), jnp.float32(0.0))

        @pl.loop(0, EB)
        def _(r):
            ones[r, :] = e0

        pltpu.sync_copy(dst_hbm.at[pl.ds(wid * GPT, GPT)], idx_d)
        plsc.subcore_barrier()

        @pl.loop(0, GPT)
        def _(j):
            pltpu.sync_copy(ones, acc.at[idx_d.at[j]], add=True)

        plsc.subcore_barrier()
        pltpu.sync_copy(acc.at[pl.ds(base, RPT)],
                        out_hbm.at[pl.ds(c * N_PAD + base, RPT)])

    return degree


def _mm_body(x_ref, w_ref, o_ref):
    h = jnp.dot(x_ref[...], w_ref[...], precision=_HIGH,
                preferred_element_type=jnp.float32)
    o_ref[0:N, :] = h
    o_ref[N:N_PAD, :] = jnp.zeros((N_PAD - N, D), jnp.float32)


def _tc_matmul_pad(x, w):
    return pl.pallas_call(
        _mm_body,
        out_shape=jax.ShapeDtypeStruct((N_PAD, D), jnp.float32),
    )(x, w)


def _scale_body(h_ref, degp_ref, hp_ref, dinv_ref):
    deg = (degp_ref[0:N_PAD, 0:1] + degp_ref[N_PAD:2 * N_PAD, 0:1]) + 1.0
    rowid = lax.broadcasted_iota(jnp.int32, (N_PAD, 1), 0)
    dinv = jnp.where(rowid < N, lax.rsqrt(deg), 0.0)
    dinv_ref[...] = dinv
    hp_ref[...] = h_ref[...] * dinv


def _tc_scale(h, degp):
    return pl.pallas_call(
        _scale_body,
        out_shape=(jax.ShapeDtypeStruct((N_PAD, D), jnp.float32),
                   jax.ShapeDtypeStruct((N_PAD, 1), jnp.float32)),
    )(h, degp)


def _post_conv(p_ref, h_ref, dinv_ref, b_ref, g_ref, be_ref):
    hp = h_ref[0:N, :]
    y = (p_ref[0:N, :] + p_ref[N_PAD:N_PAD + N, :] + hp) * dinv_ref[0:N, :]
    y = y + b_ref[...]
    y = jnp.where(y >= 0, y, NEG * y)
    mean = jnp.mean(y, axis=0, keepdims=True)
    cent = y - mean
    var = jnp.mean(cent * cent, axis=0, keepdims=True)
    return cent / jnp.sqrt(var + EPS) * g_ref[...] + be_ref[...]


def _mid_body(p_ref, h_ref, dinv_ref, b_ref, g_ref, be_ref, w_ref, o_ref):
    z = _post_conv(p_ref, h_ref, dinv_ref, b_ref, g_ref, be_ref)
    hn = jnp.dot(z, w_ref[...], precision=_HIGH,
                 preferred_element_type=jnp.float32)
    o_ref[0:N, :] = hn * dinv_ref[0:N, :]
    o_ref[N:N_PAD, :] = jnp.zeros((N_PAD - N, D), jnp.float32)


def _tc_mid(p, h, dinv, b, g, be, w_next):
    return pl.pallas_call(
        _mid_body,
        out_shape=jax.ShapeDtypeStruct((N_PAD, D), jnp.float32),
    )(p, h, dinv, b, g, be, w_next)


def _final_body(p_ref, h_ref, dinv_ref, b_ref, g_ref, be_ref, batch_ref, o_ref):
    z = _post_conv(p_ref, h_ref, dinv_ref, b_ref, g_ref, be_ref)
    onehot = (batch_ref[...] == lax.broadcasted_iota(jnp.int32, (1, G), 1))
    onehot = onehot.astype(jnp.float32)
    sums = jax.lax.dot_general(onehot, z, (((0,), (0,)), ((), ())),
                               precision=_HIGH,
                               preferred_element_type=jnp.float32)
    counts = jnp.sum(onehot, axis=0)[:, None]
    o_ref[...] = sums / jnp.maximum(counts, 1.0)


def _tc_final(p, h, dinv, b, g, be, batch2d):
    return pl.pallas_call(
        _final_body,
        out_shape=jax.ShapeDtypeStruct((G, D), jnp.float32),
    )(p, h, dinv, b, g, be, batch2d)


def kernel(x, edge_index, batch, W1, b1, g1, be1, W2, b2, g2, be2,
           W3, b3, g3, be3):
    src = edge_index[0].astype(jnp.int32)
    dst = edge_index[1].astype(jnp.int32)
    pad = jnp.full((E_PAD - E,), N, jnp.int32)
    src2d = jnp.concatenate([src, pad]).reshape(NW * GPT, EB)
    dst2d = jnp.concatenate([dst, pad]).reshape(NW * GPT, EB)
    batch2d = batch.astype(jnp.int32).reshape(N, 1)
    b1, g1, be1 = b1.reshape(1, D), g1.reshape(1, D), be1.reshape(1, D)
    b2, g2, be2 = b2.reshape(1, D), g2.reshape(1, D), be2.reshape(1, D)
    b3, g3, be3 = b3.reshape(1, D), g3.reshape(1, D), be3.reshape(1, D)

    sc_agg = _make_sc_agg()
    degp = _make_sc_degree()(dst2d)
    h1 = _tc_matmul_pad(x, W1)
    h1p, dinv = _tc_scale(h1, degp)

    p1 = sc_agg(h1p, src2d, dst2d)
    h2p = _tc_mid(p1, h1p, dinv, b1, g1, be1, W2)
    p2 = sc_agg(h2p, src2d, dst2d)
    h3p = _tc_mid(p2, h2p, dinv, b2, g2, be2, W3)
    return _tc_final(sc_agg(h3p, src2d, dst2d), h3p, dinv, b3, g3, be3, batch2d)
